# baseline (speedup 1.0000x reference)
import jax
import jax.numpy as jnp
from jax import lax
from jax.experimental import pallas as pl
from jax.experimental.pallas import tpu as pltpu
from jax.experimental.pallas import tpu_sc as plsc

ROWS = 1_000_000
COLS = 32
NC = 2
NS = 16
H = ROWS // NC
R = 248
T = -(-H // R)
G = -(-T // NS)
NBUF = 4


def _copy_body(x_hbm, o0, o1, slab, *rest):
    sin = rest[:NBUF]
    sout = rest[NBUF:]
    sid = lax.axis_index("s")
    cid = lax.axis_index("c")
    bufs = [slab.at[sid, b] for b in range(NBUF)]

    def pipeline(out_hbm, xoff):
        def base(i):
            t = jnp.minimum(sid * G + i, T - 1)
            return pl.multiple_of(jnp.minimum(t * R, H - R), 8)

        def start_in(i):
            b = i % NBUF
            pltpu.make_async_copy(
                x_hbm.at[pl.ds(xoff + base(i), R)], bufs[b], sin[b]).start()

        for g in range(NBUF):
            start_in(g)
        for g in range(G):
            b = g % NBUF
            pltpu.make_async_copy(
                x_hbm.at[pl.ds(xoff + base(g), R)], bufs[b], sin[b]).wait()
            pltpu.make_async_copy(bufs[b], out_hbm.at[pl.ds(base(g), R)], sout[b]).start()
            j = g - 1
            if j >= 0 and j + NBUF < G:
                jb = j % NBUF
                pltpu.make_async_copy(
                    bufs[jb], out_hbm.at[pl.ds(base(j), R)], sout[jb]).wait()
                start_in(j + NBUF)
        for j in range(max(0, G - NBUF), G):
            jb = j % NBUF
            pltpu.make_async_copy(
                bufs[jb], out_hbm.at[pl.ds(base(j), R)], sout[jb]).wait()

    @pl.when(cid == 0)
    def _():
        pipeline(o0, 0)

    @pl.when(cid == 1)
    def _():
        pipeline(o1, H)


@jax.jit
def kernel(x):
    o0, o1 = pl.kernel(
        _copy_body,
        out_type=[
            jax.ShapeDtypeStruct((H, COLS), jnp.float32),
            jax.ShapeDtypeStruct((H, COLS), jnp.float32),
        ],
        mesh=plsc.VectorSubcoreMesh(core_axis_name="c", subcore_axis_name="s"),
        scratch_types=(
            [pltpu.VMEM_SHARED((NS, NBUF, R, COLS), jnp.float32)]
            + [pltpu.SemaphoreType.DMA for _ in range(2 * NBUF)]
        ),
    )(x)
    return jnp.concatenate([o0, o1], axis=0)
